# Initial kernel scaffold; baseline (speedup 1.0000x reference)
import functools

import jax
import jax.numpy as jnp
from jax import lax
from jax.experimental import pallas as pl
from jax.experimental.pallas import tpu as pltpu
from jax.experimental.pallas import tpu_sc as plsc

NC = 2
NS = 16
LN = 16
NW = NC * NS
CK = 128
EPS = 1e-5

_mesh = plsc.VectorSubcoreMesh(core_axis_name="c", subcore_axis_name="s")


def _bcast16(vec, j):
    idx = jnp.full((LN,), j, dtype=jnp.int32)
    return jnp.take(vec, idx, mode=lax.GatherScatterMode.PROMISE_IN_BOUNDS)


def _deg_call(dst3, w3, zeros_deg, np_rows):
    chunks = dst3.shape[1]
    rs = np_rows // NS

    @functools.partial(
        pl.kernel,
        mesh=_mesh,
        out_type=jax.ShapeDtypeStruct((NC, np_rows, LN), jnp.float32),
        scratch_types=[
            pltpu.VMEM((chunks, CK), jnp.int32),
            pltpu.VMEM((chunks, CK), jnp.float32),
            pltpu.VMEM((CK, LN), jnp.float32),
            pltpu.VMEM_SHARED((np_rows, LN), jnp.float32),
        ],
    )
    def k(dst_hbm, w_hbm, z_hbm, out_hbm, idx_v, w_v, msg_v, acc_sh):
        cid = lax.axis_index("c")
        sid = lax.axis_index("s")
        wid = sid * NC + cid
        pltpu.sync_copy(dst_hbm.at[wid], idx_v)
        pltpu.sync_copy(w_hbm.at[wid], w_v)
        pltpu.sync_copy(z_hbm, acc_sh.at[pl.ds(sid * rs, rs)])
        plsc.subcore_barrier()

        @pl.loop(0, chunks)
        def _(g):
            for jj in range(CK // LN):
                w16 = w_v[g, pl.ds(jj * LN, LN)]
                for j2 in range(LN):
                    msg_v[jj * LN + j2, :] = _bcast16(w16, j2)
            pltpu.sync_copy(msg_v, acc_sh.at[idx_v.at[g]], add=True)

        plsc.subcore_barrier()
        pltpu.sync_copy(acc_sh.at[pl.ds(sid * rs, rs)],
                        out_hbm.at[cid, pl.ds(sid * rs, rs)])

    return k(dst3, w3, zeros_deg)


def _spmm_call(xs, src3, dst3, w3, zeros_rows, np_rows):
    chunks = src3.shape[1]
    dmodel = xs.shape[1]
    rs = np_rows // NS

    @functools.partial(
        pl.kernel,
        mesh=_mesh,
        out_type=jax.ShapeDtypeStruct((NC, np_rows, dmodel), jnp.float32),
        scratch_types=[
            pltpu.VMEM((chunks, CK), jnp.int32),
            pltpu.VMEM((chunks, CK), jnp.int32),
            pltpu.VMEM((chunks, CK), jnp.float32),
            pltpu.VMEM((CK, dmodel), jnp.float32),
            pltpu.VMEM_SHARED((np_rows, dmodel), jnp.float32),
        ],
    )
    def k(xs_hbm, src_hbm, dst_hbm, w_hbm, z_hbm, out_hbm,
          src_v, dst_v, w_v, rows_v, acc_sh):
        cid = lax.axis_index("c")
        sid = lax.axis_index("s")
        wid = sid * NC + cid
        pltpu.sync_copy(src_hbm.at[wid], src_v)
        pltpu.sync_copy(dst_hbm.at[wid], dst_v)
        pltpu.sync_copy(w_hbm.at[wid], w_v)
        pltpu.sync_copy(z_hbm, acc_sh.at[pl.ds(sid * rs, rs)])
        plsc.subcore_barrier()

        @pl.loop(0, chunks)
        def _(g):
            pltpu.sync_copy(xs_hbm.at[src_v.at[g]], rows_v)

            @pl.loop(0, CK // LN)
            def _(jj):
                w16 = w_v[g, pl.ds(jj * LN, LN)]
                for j2 in range(LN):
                    wj = _bcast16(w16, j2)
                    j = jj * LN + j2
                    for kk in range(dmodel // LN):
                        sl = pl.ds(kk * LN, LN)
                        rows_v[j, sl] = rows_v[j, sl] * wj

            pltpu.sync_copy(rows_v, acc_sh.at[dst_v.at[g]], add=True)

        plsc.subcore_barrier()
        pltpu.sync_copy(acc_sh.at[pl.ds(sid * rs, rs)],
                        out_hbm.at[cid, pl.ds(sid * rs, rs)])

    return k(xs, src3, dst3, w3, zeros_rows)


def _dot(a, b):
    return lax.dot_general(a, b, (((1,), (1,)), ((), ())),
                           precision=lax.Precision.HIGHEST,
                           preferred_element_type=jnp.float32)


def _mm_call(x, wm):
    def body(x_ref, w_ref, o_ref):
        o_ref[...] = _dot(x_ref[...], w_ref[...])

    return pl.pallas_call(
        body,
        out_shape=jax.ShapeDtypeStruct((x.shape[0], wm.shape[0]), jnp.float32),
    )(x, wm)


def _prep_call(deg_parts, xw1, n):
    def body(dp_ref, xw_ref, dinv_ref, xs_ref):
        d = dp_ref[0, :, 0:1] + dp_ref[1, :, 0:1]
        deg = d[:n] + 1.0
        dinv = lax.rsqrt(deg)
        dinv_ref[...] = dinv
        xs_ref[...] = xw_ref[...] * dinv

    return pl.pallas_call(
        body,
        out_shape=(
            jax.ShapeDtypeStruct((n, 1), jnp.float32),
            jax.ShapeDtypeStruct((n, xw1.shape[1]), jnp.float32),
        ),
    )(deg_parts, xw1)


def _mid_call(parts1, xw1, dinv, b1, gamma1, beta1, w2, n):
    def body(p_ref, xw_ref, di_ref, b1_ref, g_ref, be_ref, w2_ref,
             h1_ref, xw2_ref, xs2_ref):
        s = p_ref[0, :n, :] + p_ref[1, :n, :]
        dinv = di_ref[...]
        t = dinv * s + (dinv * dinv) * xw_ref[...] + b1_ref[...]
        mean = jnp.mean(t, axis=0, keepdims=True)
        c = t - mean
        var = jnp.mean(c * c, axis=0, keepdims=True)
        h1 = jnp.maximum(c * lax.rsqrt(var + EPS) * g_ref[...] + be_ref[...],
                         0.0)
        h1_ref[...] = h1
        xw2 = _dot(h1, w2_ref[...])
        xw2_ref[...] = xw2
        xs2_ref[...] = xw2 * dinv

    h = w2.shape[0]
    return pl.pallas_call(
        body,
        out_shape=(
            jax.ShapeDtypeStruct((n, h), jnp.float32),
            jax.ShapeDtypeStruct((n, h), jnp.float32),
            jax.ShapeDtypeStruct((n, h), jnp.float32),
        ),
    )(parts1, xw1, dinv, b1, gamma1, beta1, w2)


def _final_call(parts2, xw2, dinv, b2, h1, wf, bf, n):
    def body(p_ref, xw_ref, di_ref, b2_ref, h1_ref, wf_ref, bf_ref, o_ref):
        s = p_ref[0, :n, :] + p_ref[1, :n, :]
        dinv = di_ref[...]
        h2 = dinv * s + (dinv * dinv) * xw_ref[...] + b2_ref[...]
        hjk = jnp.maximum(h1_ref[...], h2)
        o_ref[...] = _dot(hjk, wf_ref[...]) + bf_ref[...]

    return pl.pallas_call(
        body,
        out_shape=jax.ShapeDtypeStruct((n, wf.shape[0]), jnp.float32),
    )(parts2, xw2, dinv, b2, h1, wf, bf)


def kernel(x, edge_index, edge_weight, W1, b1, gamma1, beta1, W2, b2, Wf, bf):
    n = x.shape[0]
    e = edge_index.shape[1]

    src = edge_index[0].astype(jnp.int32)
    dst = edge_index[1].astype(jnp.int32)
    w = edge_weight.astype(jnp.float32)

    chunks = -(-e // (NW * CK))
    ep = NW * chunks * CK
    pad = ep - e
    np_rows = ((n + 8) + NS - 1) // NS * NS

    pidx = jnp.arange(pad, dtype=jnp.int32)
    src_p = jnp.concatenate([src, pidx % 16])
    dst_p = jnp.concatenate([dst, n + (pidx % 8)])
    w_p = jnp.concatenate([w, jnp.zeros((pad,), jnp.float32)])
    src3 = src_p.reshape(NW, chunks, CK)
    dst3 = dst_p.reshape(NW, chunks, CK)
    w3 = w_p.reshape(NW, chunks, CK)

    rs = np_rows // NS
    zeros_deg = jnp.zeros((rs, LN), jnp.float32)
    zeros_rows = jnp.zeros((rs, x.shape[1]), jnp.float32)

    b1r = b1.reshape(1, -1)
    g1r = gamma1.reshape(1, -1)
    be1r = beta1.reshape(1, -1)
    b2r = b2.reshape(1, -1)
    bfr = bf.reshape(1, -1)

    deg_parts = _deg_call(dst3, w3, zeros_deg, np_rows)
    xw1 = _mm_call(x, W1)
    dinv, xs1 = _prep_call(deg_parts, xw1, n)
    parts1 = _spmm_call(xs1, src3, dst3, w3, zeros_rows, np_rows)
    h1, xw2, xs2 = _mid_call(parts1, xw1, dinv, b1r, g1r, be1r, W2, n)
    parts2 = _spmm_call(xs2, src3, dst3, w3, zeros_rows, np_rows)
    return _final_call(parts2, xw2, dinv, b2r, h1, Wf, bfr, n)


# speedup vs baseline: 12.8381x; 12.8381x over previous
import dataclasses
import functools

import jax
import jax.numpy as jnp
from jax import lax
from jax.experimental import pallas as pl
from jax.experimental.pallas import tpu as pltpu
from jax.experimental.pallas import tpu_sc as plsc

NC = 2
NS = 16
LN = 16
NW = NC * NS
CK = 128
EPS = 1e-5

_mesh = plsc.VectorSubcoreMesh(core_axis_name="c", subcore_axis_name="s")

_sc_params = pltpu.CompilerParams()
if "needs_layout_passes" in pltpu.CompilerParams.__dataclass_fields__:
    _sc_params = dataclasses.replace(_sc_params, needs_layout_passes=False)


def _bcast16(ref, j):
    return plsc.load_gather(ref, [jnp.full((LN,), j, dtype=jnp.int32)])


def _deg_call(dst3, w3, zeros_rows, np_rows, dmodel):
    chunks = dst3.shape[1]
    rs = np_rows // NS

    @functools.partial(
        pl.kernel,
        mesh=_mesh,
        compiler_params=_sc_params,
        out_type=jax.ShapeDtypeStruct((NC, np_rows, dmodel), jnp.float32),
        scratch_types=[
            pltpu.VMEM((chunks, CK), jnp.int32),
            pltpu.VMEM((chunks, CK), jnp.float32),
            pltpu.VMEM((CK, dmodel), jnp.float32),
            pltpu.VMEM_SHARED((np_rows, dmodel), jnp.float32),
        ],
    )
    def k(dst_hbm, w_hbm, z_hbm, out_hbm, idx_v, w_v, msg_v, acc_sh):
        cid = lax.axis_index("c")
        sid = lax.axis_index("s")
        wid = sid * NC + cid
        pltpu.sync_copy(dst_hbm.at[wid], idx_v)
        pltpu.sync_copy(w_hbm.at[wid], w_v)
        pltpu.sync_copy(z_hbm, acc_sh.at[pl.ds(sid * rs, rs)])
        plsc.subcore_barrier()

        @pl.loop(0, chunks)
        def _(g):
            @pl.loop(0, CK // LN)
            def _(jj):
                for j2 in range(LN):
                    j = jj * LN + j2
                    wj = _bcast16(w_v.at[g], j)
                    row = msg_v.at[j]
                    for kk in range(dmodel // LN):
                        row[pl.ds(kk * LN, LN)] = wj
            pltpu.sync_copy(msg_v, acc_sh.at[idx_v.at[g]], add=True)

        plsc.subcore_barrier()
        pltpu.sync_copy(acc_sh.at[pl.ds(sid * rs, rs)],
                        out_hbm.at[cid, pl.ds(sid * rs, rs)])

    return k(dst3, w3, zeros_rows)


def _spmm_call(xs, src3, dst3, w3, zeros_rows, np_rows):
    chunks = src3.shape[1]
    dmodel = xs.shape[1]
    rs = np_rows // NS

    @functools.partial(
        pl.kernel,
        mesh=_mesh,
        compiler_params=_sc_params,
        out_type=jax.ShapeDtypeStruct((NC, np_rows, dmodel), jnp.float32),
        scratch_types=[
            pltpu.VMEM((chunks, CK), jnp.int32),
            pltpu.VMEM((chunks, CK), jnp.int32),
            pltpu.VMEM((chunks, CK), jnp.float32),
            pltpu.VMEM((CK, dmodel), jnp.float32),
            pltpu.VMEM_SHARED((np_rows, dmodel), jnp.float32),
        ],
    )
    def k(xs_hbm, src_hbm, dst_hbm, w_hbm, z_hbm, out_hbm,
          src_v, dst_v, w_v, rows_v, acc_sh):
        cid = lax.axis_index("c")
        sid = lax.axis_index("s")
        wid = sid * NC + cid
        pltpu.sync_copy(src_hbm.at[wid], src_v)
        pltpu.sync_copy(dst_hbm.at[wid], dst_v)
        pltpu.sync_copy(w_hbm.at[wid], w_v)
        pltpu.sync_copy(z_hbm, acc_sh.at[pl.ds(sid * rs, rs)])
        plsc.subcore_barrier()

        @pl.loop(0, chunks)
        def _(g):
            pltpu.sync_copy(xs_hbm.at[src_v.at[g]], rows_v)

            @pl.loop(0, CK // LN)
            def _(jj):
                for j2 in range(LN):
                    wj = _bcast16(w_v.at[g], jj * LN + j2)
                    row = rows_v.at[jj * LN + j2]
                    for kk in range(dmodel // LN):
                        sl = pl.ds(kk * LN, LN)
                        row[sl] = row[sl] * wj

            pltpu.sync_copy(rows_v, acc_sh.at[dst_v.at[g]], add=True)

        plsc.subcore_barrier()
        pltpu.sync_copy(acc_sh.at[pl.ds(sid * rs, rs)],
                        out_hbm.at[cid, pl.ds(sid * rs, rs)])

    return k(xs, src3, dst3, w3, zeros_rows)


def _dot(a, b):
    return lax.dot_general(a, b, (((1,), (1,)), ((), ())),
                           precision=lax.Precision.HIGHEST,
                           preferred_element_type=jnp.float32)


def _mm_call(x, wm):
    def body(x_ref, w_ref, o_ref):
        o_ref[...] = _dot(x_ref[...], w_ref[...])

    return pl.pallas_call(
        body,
        out_shape=jax.ShapeDtypeStruct((x.shape[0], wm.shape[0]), jnp.float32),
    )(x, wm)


def _prep_call(deg_parts, xw1, n):
    def body(dp_ref, xw_ref, dinv_ref, xs_ref):
        d = dp_ref[0, :, 0:1] + dp_ref[1, :, 0:1]
        deg = d[:n] + 1.0
        dinv = lax.rsqrt(deg)
        dinv_ref[...] = dinv
        xs_ref[...] = xw_ref[...] * dinv

    return pl.pallas_call(
        body,
        out_shape=(
            jax.ShapeDtypeStruct((n, 1), jnp.float32),
            jax.ShapeDtypeStruct((n, xw1.shape[1]), jnp.float32),
        ),
    )(deg_parts, xw1)


def _mid_call(parts1, xw1, dinv, b1, gamma1, beta1, w2, n):
    h = w2.shape[0]

    def body_a(p_ref, xw_ref, di_ref, b1_ref, t_ref, mean_ref, var_ref):
        s = p_ref[0, :n, :] + p_ref[1, :n, :]
        dinv = di_ref[...]
        t = dinv * s + (dinv * dinv) * xw_ref[...] + b1_ref[...]
        t_ref[...] = t
        mean = jnp.mean(t, axis=0, keepdims=True)
        mean_ref[...] = mean
        c = t - mean
        var_ref[...] = jnp.mean(c * c, axis=0, keepdims=True)

    t, mean, var = pl.pallas_call(
        body_a,
        out_shape=(
            jax.ShapeDtypeStruct((n, h), jnp.float32),
            jax.ShapeDtypeStruct((1, h), jnp.float32),
            jax.ShapeDtypeStruct((1, h), jnp.float32),
        ),
    )(parts1, xw1, dinv, b1)

    def body_b(t_ref, mean_ref, var_ref, g_ref, be_ref, w2_ref, di_ref,
               h1_ref, xw2_ref, xs2_ref):
        c = t_ref[...] - mean_ref[...]
        h1 = jnp.maximum(
            c * lax.rsqrt(var_ref[...] + EPS) * g_ref[...] + be_ref[...], 0.0)
        h1_ref[...] = h1
        xw2 = _dot(h1, w2_ref[...])
        xw2_ref[...] = xw2
        xs2_ref[...] = xw2 * di_ref[...]

    return pl.pallas_call(
        body_b,
        out_shape=(
            jax.ShapeDtypeStruct((n, h), jnp.float32),
            jax.ShapeDtypeStruct((n, h), jnp.float32),
            jax.ShapeDtypeStruct((n, h), jnp.float32),
        ),
    )(t, mean, var, gamma1, beta1, w2, dinv)


def _final_call(parts2, xw2, dinv, b2, h1, wf, bf, n):
    def body(p_ref, xw_ref, di_ref, b2_ref, h1_ref, wf_ref, bf_ref, o_ref):
        s = p_ref[0, :n, :] + p_ref[1, :n, :]
        dinv = di_ref[...]
        h2 = dinv * s + (dinv * dinv) * xw_ref[...] + b2_ref[...]
        hjk = jnp.maximum(h1_ref[...], h2)
        o_ref[...] = _dot(hjk, wf_ref[...]) + bf_ref[...]

    return pl.pallas_call(
        body,
        out_shape=jax.ShapeDtypeStruct((n, wf.shape[0]), jnp.float32),
    )(parts2, xw2, dinv, b2, h1, wf, bf)


def kernel(x, edge_index, edge_weight, W1, b1, gamma1, beta1, W2, b2, Wf, bf):
    n = x.shape[0]
    e = edge_index.shape[1]

    src = edge_index[0].astype(jnp.int32)
    dst = edge_index[1].astype(jnp.int32)
    w = edge_weight.astype(jnp.float32)

    chunks = -(-e // (NW * CK))
    ep = NW * chunks * CK
    pad = ep - e
    np_rows = ((n + 8) + NS * 8 - 1) // (NS * 8) * (NS * 8)

    pidx = jnp.arange(pad, dtype=jnp.int32)
    src_p = jnp.concatenate([src, pidx % 16])
    dst_p = jnp.concatenate([dst, n + (pidx % 8)])
    w_p = jnp.concatenate([w, jnp.zeros((pad,), jnp.float32)])
    src3 = src_p.reshape(NW, chunks, CK)
    dst3 = dst_p.reshape(NW, chunks, CK)
    w3 = w_p.reshape(NW, chunks, CK)

    rs = np_rows // NS
    zeros_rows = jnp.zeros((rs, x.shape[1]), jnp.float32)

    b1r = b1.reshape(1, -1)
    g1r = gamma1.reshape(1, -1)
    be1r = beta1.reshape(1, -1)
    b2r = b2.reshape(1, -1)
    bfr = bf.reshape(1, -1)

    deg_parts = _deg_call(dst3, w3, zeros_rows, np_rows, x.shape[1])
    xw1 = _mm_call(x, W1)
    dinv, xs1 = _prep_call(deg_parts, xw1, n)
    parts1 = _spmm_call(xs1, src3, dst3, w3, zeros_rows, np_rows)
    h1, xw2, xs2 = _mid_call(parts1, xw1, dinv, b1r, g1r, be1r, W2, n)
    parts2 = _spmm_call(xs2, src3, dst3, w3, zeros_rows, np_rows)
    return _final_call(parts2, xw2, dinv, b2r, h1, Wf, bfr, n)


# speedup vs baseline: 19.2038x; 1.4958x over previous
import dataclasses
import functools

import jax
import jax.numpy as jnp
from jax import lax
from jax.experimental import pallas as pl
from jax.experimental.pallas import tpu as pltpu
from jax.experimental.pallas import tpu_sc as plsc

NC = 2
NS = 16
LN = 16
NW = NC * NS
CK = 112
EPS = 1e-5

_mesh = plsc.VectorSubcoreMesh(core_axis_name="c", subcore_axis_name="s")

_sc_params = pltpu.CompilerParams()
if "needs_layout_passes" in pltpu.CompilerParams.__dataclass_fields__:
    _sc_params = dataclasses.replace(_sc_params, needs_layout_passes=False)


def _bcast16(ref, j):
    v = plsc.load_gather(ref, [jnp.full((LN,), j, dtype=jnp.int32)])
    return plsc.bitcast(v, jnp.float32)


def _copy_row(src_ref, dst_ref):
    for kk in range(CK // LN):
        sl = pl.ds(kk * LN, LN)
        dst_ref[sl] = src_ref[sl]


def _deg_call(esd, zeros_rows, np_rows, dmodel):
    chunks = esd.shape[1]
    rs = np_rows // NS

    @functools.partial(
        pl.kernel,
        mesh=_mesh,
        compiler_params=_sc_params,
        out_type=jax.ShapeDtypeStruct((NC, np_rows, dmodel), jnp.float32),
        scratch_types=[
            pltpu.VMEM((3, 8, CK), jnp.int32),
            pltpu.VMEM((2, CK), jnp.int32),
            pltpu.VMEM((CK, dmodel), jnp.float32),
            pltpu.VMEM((CK, dmodel), jnp.float32),
            pltpu.VMEM_SHARED((np_rows, dmodel), jnp.float32),
            pltpu.SemaphoreType.DMA((3,)),
            pltpu.SemaphoreType.DMA((2,)),
        ],
    )
    def k(esd_hbm, z_hbm, out_hbm, esd_r, dst_b, msg_v, msg2_v, acc_sh,
          isems, ssems):
        cid = lax.axis_index("c")
        sid = lax.axis_index("s")
        wid = sid * NC + cid
        pltpu.sync_copy(z_hbm, acc_sh.at[pl.ds(sid * rs, rs)])
        plsc.subcore_barrier()

        msgs = [msg_v, msg2_v]

        def i_issue(g, r):
            pltpu.async_copy(esd_hbm.at[wid, g], esd_r.at[r], isems.at[r])

        def i_wait(g, r):
            pltpu.make_async_copy(esd_hbm.at[wid, g], esd_r.at[r],
                                  isems.at[r]).wait()

        def build(g, r, bb):
            w_ref = esd_r.at[r, 2]

            @pl.loop(0, CK // LN)
            def _(jj):
                for j2 in range(LN):
                    j = jj * LN + j2
                    wj = _bcast16(w_ref, j)
                    row = msgs[bb].at[j]
                    for kk in range(dmodel // LN):
                        row[pl.ds(kk * LN, LN)] = wj

        def s_issue(g, bb):
            pltpu.async_copy(msgs[bb], acc_sh.at[dst_b.at[bb]], ssems.at[bb],
                             add=True)

        def s_wait(g, bb):
            pltpu.make_async_copy(msgs[bb], acc_sh.at[dst_b.at[bb]],
                                  ssems.at[bb]).wait()

        def chunk(g, bb, r, do_swait, do_iissue):
            if do_swait:
                s_wait(g - 2, bb)
            i_wait(g, r)
            build(g, r, bb)
            _copy_row(esd_r.at[r, 1], dst_b.at[bb])
            s_issue(g, bb)
            if do_iissue:
                i_issue(g + 3, r)

        for r in range(3):
            i_issue(r, r)
        for g in range(6):
            chunk(g, g % 2, g % 3, g >= 2, True)

        @pl.loop(1, chunks // 6 - 1)
        def _(t):
            for u in range(6):
                chunk(t * 6 + u, u % 2, u % 3, True, True)

        for u in range(6):
            g = chunks - 6 + u
            chunk(g, u % 2, u % 3, True, g + 3 < chunks)

        s_wait(chunks - 2, 0)
        s_wait(chunks - 1, 1)
        plsc.subcore_barrier()
        pltpu.sync_copy(acc_sh.at[pl.ds(sid * rs, rs)],
                        out_hbm.at[cid, pl.ds(sid * rs, rs)])

    return k(esd, zeros_rows)


def _spmm_call(xs, esd, zeros_rows, np_rows):
    chunks = esd.shape[1]
    dmodel = xs.shape[1]
    rs = np_rows // NS

    @functools.partial(
        pl.kernel,
        mesh=_mesh,
        compiler_params=_sc_params,
        out_type=jax.ShapeDtypeStruct((NC, np_rows, dmodel), jnp.float32),
        scratch_types=[
            pltpu.VMEM((3, 8, CK), jnp.int32),
            pltpu.VMEM((3, CK), jnp.int32),
            pltpu.VMEM((CK, dmodel), jnp.float32),
            pltpu.VMEM((CK, dmodel), jnp.float32),
            pltpu.VMEM((CK, dmodel), jnp.float32),
            pltpu.VMEM_SHARED((np_rows, dmodel), jnp.float32),
            pltpu.SemaphoreType.DMA((3,)),
            pltpu.SemaphoreType.DMA((3,)),
            pltpu.SemaphoreType.DMA((3,)),
        ],
    )
    def k(xs_hbm, esd_hbm, z_hbm, out_hbm,
          esd_r, dst_b, rb0, rb1, rb2, acc_sh, isems, gsems, ssems):
        cid = lax.axis_index("c")
        sid = lax.axis_index("s")
        wid = sid * NC + cid
        pltpu.sync_copy(z_hbm, acc_sh.at[pl.ds(sid * rs, rs)])
        plsc.subcore_barrier()

        bufs = [rb0, rb1, rb2]

        def i_issue(g, r):
            pltpu.async_copy(esd_hbm.at[wid, g], esd_r.at[r], isems.at[r])

        def i_wait(g, r):
            pltpu.make_async_copy(esd_hbm.at[wid, g], esd_r.at[r],
                                  isems.at[r]).wait()

        def scale(g, r, bb):
            w_ref = esd_r.at[r, 2]

            @pl.loop(0, CK // LN)
            def _(jj):
                for j2 in range(LN):
                    j = jj * LN + j2
                    wj = _bcast16(w_ref, j)
                    row = bufs[bb].at[j]
                    for kk in range(dmodel // LN):
                        sl = pl.ds(kk * LN, LN)
                        row[sl] = row[sl] * wj

        def g_issue(g, r, bb):
            pltpu.async_copy(xs_hbm.at[esd_r.at[r, 0]], bufs[bb],
                             gsems.at[bb])

        def g_wait(g, r, bb):
            pltpu.make_async_copy(xs_hbm.at[esd_r.at[r, 0]], bufs[bb],
                                  gsems.at[bb]).wait()

        def s_issue(g, bb):
            pltpu.async_copy(bufs[bb], acc_sh.at[dst_b.at[bb]], ssems.at[bb],
                             add=True)

        def s_wait(g, bb):
            pltpu.make_async_copy(bufs[bb], acc_sh.at[dst_b.at[bb]],
                                  ssems.at[bb]).wait()

        def chunk(g, bb, do_swait, do_gissue, do_iissue):
            r = bb
            g_wait(g, r, bb)
            scale(g, r, bb)
            _copy_row(esd_r.at[r, 1], dst_b.at[bb])
            s_issue(g, bb)
            if do_swait:
                s_wait(g - 1, (bb + 2) % 3)
            if do_gissue:
                i_wait(g + 2, (bb + 2) % 3)
                g_issue(g + 2, (bb + 2) % 3, (bb + 2) % 3)
            if do_iissue:
                i_issue(g + 3, bb)

        for r in range(3):
            i_issue(r, r)
        i_wait(0, 0)
        g_issue(0, 0, 0)
        i_wait(1, 1)
        g_issue(1, 1, 1)
        chunk(0, 0, False, True, True)
        chunk(1, 1, True, True, True)
        chunk(2, 2, True, True, True)

        @pl.loop(1, chunks // 3 - 1)
        def _(t):
            for bb in range(3):
                chunk(t * 3 + bb, bb, True, True, True)

        for u in range(3):
            g = chunks - 3 + u
            chunk(g, u, True, g + 2 < chunks, False)

        s_wait(chunks - 1, (chunks - 1) % 3)
        plsc.subcore_barrier()
        pltpu.sync_copy(acc_sh.at[pl.ds(sid * rs, rs)],
                        out_hbm.at[cid, pl.ds(sid * rs, rs)])

    return k(xs, esd, zeros_rows)


def _dot(a, b):
    return lax.dot_general(a, b, (((1,), (1,)), ((), ())),
                           precision=lax.Precision.HIGHEST,
                           preferred_element_type=jnp.float32)


def _mm_call(x, wm):
    def body(x_ref, w_ref, o_ref):
        o_ref[...] = _dot(x_ref[...], w_ref[...])

    return pl.pallas_call(
        body,
        out_shape=jax.ShapeDtypeStruct((x.shape[0], wm.shape[0]), jnp.float32),
    )(x, wm)


def _prep_call(deg_parts, xw1, n):
    def body(dp_ref, xw_ref, dinv_ref, xs_ref):
        deg = dp_ref[0, :n, 0:1] + dp_ref[1, :n, 0:1] + 1.0
        dinv = lax.rsqrt(deg)
        dinv_ref[...] = dinv
        xs_ref[...] = xw_ref[...] * dinv

    return pl.pallas_call(
        body,
        out_shape=(
            jax.ShapeDtypeStruct((n, 1), jnp.float32),
            jax.ShapeDtypeStruct((n, xw1.shape[1]), jnp.float32),
        ),
    )(deg_parts, xw1)


def _mid_call(parts1, xw1, dinv, b1, gamma1, beta1, w2, n):
    h = w2.shape[0]

    def body_a(p_ref, xw_ref, di_ref, b1_ref, t_ref, mean_ref, var_ref):
        s = p_ref[0, :n, :] + p_ref[1, :n, :]
        dinv = di_ref[...]
        t = dinv * s + (dinv * dinv) * xw_ref[...] + b1_ref[...]
        t_ref[...] = t
        mean = jnp.mean(t, axis=0, keepdims=True)
        mean_ref[...] = mean
        c = t - mean
        var_ref[...] = jnp.mean(c * c, axis=0, keepdims=True)

    t, mean, var = pl.pallas_call(
        body_a,
        out_shape=(
            jax.ShapeDtypeStruct((n, h), jnp.float32),
            jax.ShapeDtypeStruct((1, h), jnp.float32),
            jax.ShapeDtypeStruct((1, h), jnp.float32),
        ),
    )(parts1, xw1, dinv, b1)

    def body_b(t_ref, mean_ref, var_ref, g_ref, be_ref, w2_ref, di_ref,
               h1_ref, xw2_ref, xs2_ref):
        c = t_ref[...] - mean_ref[...]
        h1 = jnp.maximum(
            c * lax.rsqrt(var_ref[...] + EPS) * g_ref[...] + be_ref[...], 0.0)
        h1_ref[...] = h1
        xw2 = _dot(h1, w2_ref[...])
        xw2_ref[...] = xw2
        xs2_ref[...] = xw2 * di_ref[...]

    return pl.pallas_call(
        body_b,
        out_shape=(
            jax.ShapeDtypeStruct((n, h), jnp.float32),
            jax.ShapeDtypeStruct((n, h), jnp.float32),
            jax.ShapeDtypeStruct((n, h), jnp.float32),
        ),
    )(t, mean, var, gamma1, beta1, w2, dinv)


def _final_call(parts2, xw2, dinv, b2, h1, wf, bf, n):
    def body(p_ref, xw_ref, di_ref, b2_ref, h1_ref, wf_ref, bf_ref, o_ref):
        s = p_ref[0, :n, :] + p_ref[1, :n, :]
        dinv = di_ref[...]
        h2 = dinv * s + (dinv * dinv) * xw_ref[...] + b2_ref[...]
        hjk = jnp.maximum(h1_ref[...], h2)
        o_ref[...] = _dot(hjk, wf_ref[...]) + bf_ref[...]

    return pl.pallas_call(
        body,
        out_shape=jax.ShapeDtypeStruct((n, wf.shape[0]), jnp.float32),
    )(parts2, xw2, dinv, b2, h1, wf, bf)


def kernel(x, edge_index, edge_weight, W1, b1, gamma1, beta1, W2, b2, Wf, bf):
    n = x.shape[0]
    e = edge_index.shape[1]

    src = edge_index[0].astype(jnp.int32)
    dst = edge_index[1].astype(jnp.int32)
    w = edge_weight.astype(jnp.float32)

    chunks = -(-e // (NW * CK))
    chunks = max(12, (chunks + 5) // 6 * 6)
    ep = NW * chunks * CK
    pad = ep - e
    np_rows = ((n + 8) + NS * 8 - 1) // (NS * 8) * (NS * 8)

    pidx = jnp.arange(pad, dtype=jnp.int32)
    src_p = jnp.concatenate([src, pidx % 16])
    dst_p = jnp.concatenate([dst, n + (pidx % 8)])
    w_p = jnp.concatenate([w, jnp.zeros((pad,), jnp.float32)])
    src3 = src_p.reshape(NW, chunks, CK)
    dst3 = dst_p.reshape(NW, chunks, CK)
    wb3 = lax.bitcast_convert_type(w_p, jnp.int32).reshape(NW, chunks, CK)
    z3 = jnp.zeros_like(src3)
    esd = jnp.stack([src3, dst3, wb3, z3, z3, z3, z3, z3], axis=2)

    rs = np_rows // NS
    zeros_rows = jnp.zeros((rs, x.shape[1]), jnp.float32)

    b1r = b1.reshape(1, -1)
    g1r = gamma1.reshape(1, -1)
    be1r = beta1.reshape(1, -1)
    b2r = b2.reshape(1, -1)
    bfr = bf.reshape(1, -1)

    deg_parts = _deg_call(esd, zeros_rows, np_rows, x.shape[1])
    xw1 = _mm_call(x, W1)
    dinv, xs1 = _prep_call(deg_parts, xw1, n)
    parts1 = _spmm_call(xs1, esd, zeros_rows, np_rows)
    h1, xw2, xs2 = _mid_call(parts1, xw1, dinv, b1r, g1r, be1r, W2, n)
    parts2 = _spmm_call(xs2, esd, zeros_rows, np_rows)
    return _final_call(parts2, xw2, dinv, b2r, h1, Wf, bfr, n)


# speedup vs baseline: 19.4395x; 1.0123x over previous
import dataclasses
import functools

import jax
import jax.numpy as jnp
from jax import lax
from jax.experimental import pallas as pl
from jax.experimental.pallas import tpu as pltpu
from jax.experimental.pallas import tpu_sc as plsc

NC = 2
NS = 16
LN = 16
NW = NC * NS
CK = 112
EPS = 1e-5

_mesh = plsc.VectorSubcoreMesh(core_axis_name="c", subcore_axis_name="s")

_sc_params = pltpu.CompilerParams()
if "needs_layout_passes" in pltpu.CompilerParams.__dataclass_fields__:
    _sc_params = dataclasses.replace(_sc_params, needs_layout_passes=False)


def _bcast16(ref, j):
    v = plsc.load_gather(ref, [jnp.full((LN,), j, dtype=jnp.int32)])
    return plsc.bitcast(v, jnp.float32)


def _copy_row(src_ref, dst_ref):
    for kk in range(CK // LN):
        sl = pl.ds(kk * LN, LN)
        dst_ref[sl] = src_ref[sl]


def _deg_call(esd, zeros_rows, np_rows, dmodel):
    chunks = esd.shape[1]
    rs = np_rows // NS

    @functools.partial(
        pl.kernel,
        mesh=_mesh,
        compiler_params=_sc_params,
        out_type=jax.ShapeDtypeStruct((NC, np_rows, dmodel), jnp.float32),
        scratch_types=[
            pltpu.VMEM((3, 8, CK), jnp.int32),
            pltpu.VMEM((2, CK), jnp.int32),
            pltpu.VMEM((CK, dmodel), jnp.float32),
            pltpu.VMEM((CK, dmodel), jnp.float32),
            pltpu.VMEM_SHARED((np_rows, dmodel), jnp.float32),
            pltpu.SemaphoreType.DMA((3,)),
            pltpu.SemaphoreType.DMA((2,)),
        ],
    )
    def k(esd_hbm, z_hbm, out_hbm, esd_r, dst_b, msg_v, msg2_v, acc_sh,
          isems, ssems):
        cid = lax.axis_index("c")
        sid = lax.axis_index("s")
        wid = sid * NC + cid
        pltpu.sync_copy(z_hbm, acc_sh.at[pl.ds(sid * rs, rs)])
        plsc.subcore_barrier()

        msgs = [msg_v, msg2_v]

        def i_issue(g, r):
            pltpu.async_copy(esd_hbm.at[wid, g], esd_r.at[r], isems.at[r])

        def i_wait(g, r):
            pltpu.make_async_copy(esd_hbm.at[wid, g], esd_r.at[r],
                                  isems.at[r]).wait()

        def build(g, r, bb):
            w_ref = esd_r.at[r, 2]

            @pl.loop(0, CK // LN)
            def _(jj):
                for j2 in range(LN):
                    j = jj * LN + j2
                    wj = _bcast16(w_ref, j)
                    row = msgs[bb].at[j]
                    for kk in range(dmodel // LN):
                        row[pl.ds(kk * LN, LN)] = wj

        def s_issue(g, bb):
            pltpu.async_copy(msgs[bb], acc_sh.at[dst_b.at[bb]], ssems.at[bb],
                             add=True)

        def s_wait(g, bb):
            pltpu.make_async_copy(msgs[bb], acc_sh.at[dst_b.at[bb]],
                                  ssems.at[bb]).wait()

        def chunk(g, bb, r, do_swait, do_iissue):
            if do_swait:
                s_wait(g - 2, bb)
            i_wait(g, r)
            build(g, r, bb)
            _copy_row(esd_r.at[r, 1], dst_b.at[bb])
            s_issue(g, bb)
            if do_iissue:
                i_issue(g + 3, r)

        for r in range(3):
            i_issue(r, r)
        for g in range(6):
            chunk(g, g % 2, g % 3, g >= 2, True)

        @pl.loop(1, chunks // 6 - 1)
        def _(t):
            for u in range(6):
                chunk(t * 6 + u, u % 2, u % 3, True, True)

        for u in range(6):
            g = chunks - 6 + u
            chunk(g, u % 2, u % 3, True, g + 3 < chunks)

        s_wait(chunks - 2, 0)
        s_wait(chunks - 1, 1)
        plsc.subcore_barrier()
        pltpu.sync_copy(acc_sh.at[pl.ds(sid * rs, rs)],
                        out_hbm.at[cid, pl.ds(sid * rs, rs)])

    return k(esd, zeros_rows)


def _spmm_call(xs, esd, zeros_rows, np_rows):
    chunks = esd.shape[1]
    dmodel = xs.shape[1]
    rs = np_rows // NS

    @functools.partial(
        pl.kernel,
        mesh=_mesh,
        compiler_params=_sc_params,
        out_type=jax.ShapeDtypeStruct((NC, np_rows, dmodel), jnp.float32),
        scratch_types=[
            pltpu.VMEM((3, 8, CK), jnp.int32),
            pltpu.VMEM((3, CK), jnp.int32),
            pltpu.VMEM((CK, dmodel), jnp.float32),
            pltpu.VMEM((CK, dmodel), jnp.float32),
            pltpu.VMEM((CK, dmodel), jnp.float32),
            pltpu.VMEM_SHARED((np_rows, dmodel), jnp.float32),
            pltpu.SemaphoreType.DMA((3,)),
            pltpu.SemaphoreType.DMA((3,)),
            pltpu.SemaphoreType.DMA((3,)),
        ],
    )
    def k(xs_hbm, esd_hbm, z_hbm, out_hbm,
          esd_r, dst_b, rb0, rb1, rb2, acc_sh, isems, gsems, ssems):
        cid = lax.axis_index("c")
        sid = lax.axis_index("s")
        wid = sid * NC + cid
        pltpu.sync_copy(z_hbm, acc_sh.at[pl.ds(sid * rs, rs)])
        plsc.subcore_barrier()

        bufs = [rb0, rb1, rb2]

        def i_issue(g, r):
            pltpu.async_copy(esd_hbm.at[wid, g], esd_r.at[r], isems.at[r])

        def i_wait(g, r):
            pltpu.make_async_copy(esd_hbm.at[wid, g], esd_r.at[r],
                                  isems.at[r]).wait()

        def scale(g, r, bb):
            w_ref = esd_r.at[r, 2]

            @pl.loop(0, CK // LN)
            def _(jj):
                for j2 in range(LN):
                    j = jj * LN + j2
                    wj = _bcast16(w_ref, j)
                    row = bufs[bb].at[j]
                    for kk in range(dmodel // LN):
                        sl = pl.ds(kk * LN, LN)
                        row[sl] = row[sl] * wj

        def g_issue(g, r, bb):
            pltpu.async_copy(xs_hbm.at[esd_r.at[r, 0]], bufs[bb],
                             gsems.at[bb])

        def g_wait(g, r, bb):
            pltpu.make_async_copy(xs_hbm.at[esd_r.at[r, 0]], bufs[bb],
                                  gsems.at[bb]).wait()

        def s_issue(g, bb):
            pltpu.async_copy(bufs[bb], acc_sh.at[dst_b.at[bb]], ssems.at[bb],
                             add=True)

        def s_wait(g, bb):
            pltpu.make_async_copy(bufs[bb], acc_sh.at[dst_b.at[bb]],
                                  ssems.at[bb]).wait()

        def chunk(g, bb, do_swait, do_gissue, do_iissue):
            r = bb
            g_wait(g, r, bb)
            scale(g, r, bb)
            _copy_row(esd_r.at[r, 1], dst_b.at[bb])
            s_issue(g, bb)
            if do_swait:
                s_wait(g - 1, (bb + 2) % 3)
            if do_gissue:
                i_wait(g + 2, (bb + 2) % 3)
                g_issue(g + 2, (bb + 2) % 3, (bb + 2) % 3)
            if do_iissue:
                i_issue(g + 3, bb)

        for r in range(3):
            i_issue(r, r)
        i_wait(0, 0)
        g_issue(0, 0, 0)
        i_wait(1, 1)
        g_issue(1, 1, 1)
        chunk(0, 0, False, True, True)
        chunk(1, 1, True, True, True)
        chunk(2, 2, True, True, True)

        @pl.loop(1, chunks // 3 - 1)
        def _(t):
            for bb in range(3):
                chunk(t * 3 + bb, bb, True, True, True)

        for u in range(3):
            g = chunks - 3 + u
            chunk(g, u, True, g + 2 < chunks, False)

        s_wait(chunks - 1, (chunks - 1) % 3)
        plsc.subcore_barrier()
        pltpu.sync_copy(acc_sh.at[pl.ds(sid * rs, rs)],
                        out_hbm.at[cid, pl.ds(sid * rs, rs)])

    return k(xs, esd, zeros_rows)


def _dot(a, b):
    return lax.dot_general(a, b, (((1,), (1,)), ((), ())),
                           precision=lax.Precision.HIGHEST,
                           preferred_element_type=jnp.float32)


_VMEM_LIMIT = pltpu.CompilerParams(vmem_limit_bytes=67108864)


def _mm_call(x, wm):
    def body(x_ref, w_ref, o_ref):
        o_ref[...] = _dot(x_ref[...], w_ref[...])

    return pl.pallas_call(
        body,
        out_shape=jax.ShapeDtypeStruct((x.shape[0], wm.shape[0]), jnp.float32),
    )(x, wm)


def _prep_call(deg_parts, xw1, n):
    def body(dp_ref, xw_ref, dinv_ref, xs_ref):
        deg = dp_ref[0, :n, 0:1] + dp_ref[1, :n, 0:1] + 1.0
        dinv = lax.rsqrt(deg)
        dinv_ref[...] = dinv
        xs_ref[...] = xw_ref[...] * dinv

    return pl.pallas_call(
        body,
        out_shape=(
            jax.ShapeDtypeStruct((n, 1), jnp.float32),
            jax.ShapeDtypeStruct((n, xw1.shape[1]), jnp.float32),
        ),
    )(deg_parts, xw1)


def _mid_call(parts1, xw1, dinv, b1, gamma1, beta1, w2, n):
    h = w2.shape[0]

    def body(p_ref, xw_ref, di_ref, b1_ref, g_ref, be_ref, w2_ref,
             h1_ref, xw2_ref, xs2_ref):
        s = p_ref[0, :n, :] + p_ref[1, :n, :]
        dinv = di_ref[...]
        t = dinv * s + (dinv * dinv) * xw_ref[...] + b1_ref[...]
        mean = jnp.mean(t, axis=0, keepdims=True)
        c = t - mean
        var = jnp.mean(c * c, axis=0, keepdims=True)
        h1 = jnp.maximum(
            c * lax.rsqrt(var + EPS) * g_ref[...] + be_ref[...], 0.0)
        h1_ref[...] = h1
        xw2 = _dot(h1, w2_ref[...])
        xw2_ref[...] = xw2
        xs2_ref[...] = xw2 * dinv

    return pl.pallas_call(
        body,
        compiler_params=_VMEM_LIMIT,
        out_shape=(
            jax.ShapeDtypeStruct((n, h), jnp.float32),
            jax.ShapeDtypeStruct((n, h), jnp.float32),
            jax.ShapeDtypeStruct((n, h), jnp.float32),
        ),
    )(parts1, xw1, dinv, b1, gamma1, beta1, w2)


def _final_call(parts2, xw2, dinv, b2, h1, wf, bf, n):
    def body(p_ref, xw_ref, di_ref, b2_ref, h1_ref, wf_ref, bf_ref, o_ref):
        s = p_ref[0, :n, :] + p_ref[1, :n, :]
        dinv = di_ref[...]
        h2 = dinv * s + (dinv * dinv) * xw_ref[...] + b2_ref[...]
        hjk = jnp.maximum(h1_ref[...], h2)
        o_ref[...] = _dot(hjk, wf_ref[...]) + bf_ref[...]

    return pl.pallas_call(
        body,
        out_shape=jax.ShapeDtypeStruct((n, wf.shape[0]), jnp.float32),
    )(parts2, xw2, dinv, b2, h1, wf, bf)


def kernel(x, edge_index, edge_weight, W1, b1, gamma1, beta1, W2, b2, Wf, bf):
    n = x.shape[0]
    e = edge_index.shape[1]

    src = edge_index[0].astype(jnp.int32)
    dst = edge_index[1].astype(jnp.int32)
    w = edge_weight.astype(jnp.float32)

    chunks = -(-e // (NW * CK))
    chunks = max(12, (chunks + 5) // 6 * 6)
    ep = NW * chunks * CK
    pad = ep - e
    np_rows = ((n + 8) + NS * 8 - 1) // (NS * 8) * (NS * 8)

    pidx = jnp.arange(pad, dtype=jnp.int32)
    src_p = jnp.concatenate([src, pidx % 16])
    dst_p = jnp.concatenate([dst, n + (pidx % 8)])
    w_p = jnp.concatenate([w, jnp.zeros((pad,), jnp.float32)])
    src3 = src_p.reshape(NW, chunks, CK)
    dst3 = dst_p.reshape(NW, chunks, CK)
    wb3 = lax.bitcast_convert_type(w_p, jnp.int32).reshape(NW, chunks, CK)
    z3 = jnp.zeros_like(src3)
    esd = jnp.stack([src3, dst3, wb3, z3, z3, z3, z3, z3], axis=2)

    rs = np_rows // NS
    zeros_rows = jnp.zeros((rs, x.shape[1]), jnp.float32)

    b1r = b1.reshape(1, -1)
    g1r = gamma1.reshape(1, -1)
    be1r = beta1.reshape(1, -1)
    b2r = b2.reshape(1, -1)
    bfr = bf.reshape(1, -1)

    deg_parts = _deg_call(esd, zeros_rows, np_rows, x.shape[1])
    xw1 = _mm_call(x, W1)
    dinv, xs1 = _prep_call(deg_parts, xw1, n)
    parts1 = _spmm_call(xs1, esd, zeros_rows, np_rows)
    h1, xw2, xs2 = _mid_call(parts1, xw1, dinv, b1r, g1r, be1r, W2, n)
    parts2 = _spmm_call(xs2, esd, zeros_rows, np_rows)
    return _final_call(parts2, xw2, dinv, b2r, h1, Wf, bfr, n)
